# Initial kernel scaffold; baseline (speedup 1.0000x reference)
import jax
import jax.numpy as jnp
from jax.experimental import pallas as pl


def kernel(Graph, norm_h, norm_L, norm_W, norm_P, norm_N, T, ln_g, ln_b, W0, We0, al0, ar0, ae0, b0, W1, We1, al1, ar1, ae1, b1, Wl, bl):
    raise NotImplementedError("write your pallas kernel here")


# speedup vs baseline: 3.9934x; 3.9934x over previous
import jax
import jax.numpy as jnp
from jax import lax
from jax.experimental import pallas as pl

_J = 100
_M = 20
_N = 120
_H = 3
_F0 = 16
_ED = 128
_BB = 8


def _lrelu(x, s):
    return jnp.where(x >= 0, x, s * x)


def _gat_heads(mask, Tm, ft_all, al, ar, eec, few, brow, D):
    acc = None
    for hh in range(_H):
        ft = ft_all[:, :, hh * D:(hh + 1) * D]
        el = jnp.sum(ft * al[hh][None, None, :], axis=-1)
        er = jnp.sum(ft * ar[hh][None, None, :], axis=-1)
        ee = Tm * eec[0:1, hh:hh + 1][:, :, None]
        logits = _lrelu(el[:, :, None] + er[:, None, :] + ee, 0.2)
        logits = jnp.where(mask, logits, -1e9)
        mx = jnp.max(logits, axis=1, keepdims=True)
        ex = jnp.where(mask, jnp.exp(logits - mx), 0.0)
        den = jnp.sum(ex, axis=1, keepdims=True)
        alpha = ex / jnp.where(den > 0, den, 1.0)
        out = lax.dot_general(
            alpha, ft,
            dimension_numbers=(((1,), (1,)), ((0,), (0,))),
            preferred_element_type=jnp.float32)
        eagg = jnp.sum(alpha * Tm, axis=1)
        o = out + eagg[:, :, None] * few[hh][None, None, :] + brow[hh][None, None, :]
        o = _lrelu(o, 0.01)
        acc = o if acc is None else acc + o
    return acc * (1.0 / _H)


def _gnn_body(A_ref, T_ref, h_ref, L_ref, wpn_ref,
              W0g_ref, c0_ref, al0_ref, ar0_ref, eec0_ref, few0_ref, b0_ref,
              W1_ref, al1_ref, ar1_ref, eec1_ref, few1_ref, b1_ref,
              h1_ref):
    A = A_ref[...]
    Tm = T_ref[...]
    mask = A > 0.0

    nm = (lax.broadcasted_iota(jnp.int32, A.shape[:2], 1) < _J).astype(jnp.float32)
    wpn = wpn_ref[...]
    f0 = h_ref[...]
    f1 = L_ref[...]
    f2 = wpn[:, 0:1] * nm
    f3 = wpn[:, 1:2] * nm
    f4 = wpn[:, 2:3] * nm
    mu = (f0 + f1 + f2 + f3 + f4) * 0.2
    d0, d1, d2, d3, d4 = f0 - mu, f1 - mu, f2 - mu, f3 - mu, f4 - mu
    var = (d0 * d0 + d1 * d1 + d2 * d2 + d3 * d3 + d4 * d4) * 0.2
    rstd = lax.rsqrt(var + 1e-5)

    W0g = W0g_ref[...]
    ft0 = jnp.broadcast_to(c0_ref[...][None], A.shape[:2] + (_H * _F0,))
    for i, d in enumerate((d0, d1, d2, d3, d4)):
        ft0 = ft0 + (d * rstd)[:, :, None] * W0g[i][None, None, :]

    h0 = _gat_heads(mask, Tm, ft0,
                    al0_ref[...], ar0_ref[...], eec0_ref[...],
                    few0_ref[...], b0_ref[...], _F0)

    ft1 = lax.dot_general(h0, W1_ref[...],
                          dimension_numbers=(((2,), (0,)), ((), ())),
                          preferred_element_type=jnp.float32)

    h1 = _gat_heads(mask, Tm, ft1,
                    al1_ref[...], ar1_ref[...], eec1_ref[...],
                    few1_ref[...], b1_ref[...], _ED)
    h1_ref[...] = h1


def _final_body(x_ref, w_ref, bl_ref, o_ref, *, kb):
    k = pl.program_id(0)
    acc = jnp.dot(x_ref[...], w_ref[...], preferred_element_type=jnp.float32)

    @pl.when(k == 0)
    def _():
        o_ref[...] = acc

    @pl.when(k > 0)
    def _():
        o_ref[...] = o_ref[...] + acc

    @pl.when(k == kb - 1)
    def _():
        o_ref[...] = _lrelu(o_ref[...] + bl_ref[...], 0.01)


def kernel(Graph, norm_h, norm_L, norm_W, norm_P, norm_N, T, ln_g, ln_b,
           W0, We0, al0, ar0, ae0, b0, W1, We1, al1, ar1, ae1, b1, Wl, bl):
    BS = Graph.shape[0]
    G = Graph.reshape(BS, _J, _N)
    A = jnp.zeros((BS, _N, _N), jnp.float32).at[:, :_J, :].set(G)
    Tm = jnp.zeros((BS, _N, _N), jnp.float32).at[:, :_J, :_J].set(T)
    h_pad = jnp.pad(norm_h, ((0, 0), (0, _M)))
    L_pad = jnp.pad(norm_L, ((0, 0), (0, _M)))
    wpn = jnp.concatenate([norm_W, norm_P, norm_N], axis=-1)

    W0g = W0 * ln_g[:, None]
    c0 = (ln_b @ W0).reshape(1, _H * _F0)
    few0 = We0.reshape(_H, _F0)
    eec0 = jnp.sum(few0 * ae0, axis=-1).reshape(1, _H)
    b0r = b0.reshape(_H, _F0)
    few1 = We1.reshape(_H, _ED)
    eec1 = jnp.sum(few1 * ae1, axis=-1).reshape(1, _H)
    b1r = b1.reshape(_H, _ED)

    nsteps = BS // _BB
    const = lambda *shape: pl.BlockSpec(shape, lambda i: (0,) * len(shape))
    h1 = pl.pallas_call(
        _gnn_body,
        grid=(nsteps,),
        in_specs=[
            pl.BlockSpec((_BB, _N, _N), lambda i: (i, 0, 0)),
            pl.BlockSpec((_BB, _N, _N), lambda i: (i, 0, 0)),
            pl.BlockSpec((_BB, _N), lambda i: (i, 0)),
            pl.BlockSpec((_BB, _N), lambda i: (i, 0)),
            pl.BlockSpec((_BB, 3), lambda i: (i, 0)),
            const(5, _H * _F0),
            const(1, _H * _F0),
            const(_H, _F0), const(_H, _F0),
            const(1, _H),
            const(_H, _F0), const(_H, _F0),
            const(_F0, _H * _ED),
            const(_H, _ED), const(_H, _ED),
            const(1, _H),
            const(_H, _ED), const(_H, _ED),
        ],
        out_specs=pl.BlockSpec((_BB, _N, _ED), lambda i: (i, 0, 0)),
        out_shape=jax.ShapeDtypeStruct((BS, _N, _ED), jnp.float32),
    )(A, Tm, h_pad, L_pad, wpn, W0g, c0, al0, ar0, eec0, few0, b0r,
      W1, al1, ar1, eec1, few1, b1r)

    X2 = h1.reshape(BS, _N * _ED)
    KB = 8
    BK = (_N * _ED) // KB
    import functools
    out = pl.pallas_call(
        functools.partial(_final_body, kb=KB),
        grid=(KB,),
        in_specs=[
            pl.BlockSpec((BS, BK), lambda k: (0, k)),
            pl.BlockSpec((BK, _ED), lambda k: (k, 0)),
            pl.BlockSpec((1, _ED), lambda k: (0, 0)),
        ],
        out_specs=pl.BlockSpec((BS, _ED), lambda k: (0, 0)),
        out_shape=jax.ShapeDtypeStruct((BS, _ED), jnp.float32),
    )(X2, Wl, bl.reshape(1, _ED))
    return out


# speedup vs baseline: 4.2225x; 1.0574x over previous
import jax
import jax.numpy as jnp
from jax import lax
from jax.experimental import pallas as pl

_J = 100
_M = 20
_N = 120
_H = 3
_F0 = 16
_ED = 128
_BB = 8


def _lrelu(x, s):
    return jnp.where(x >= 0, x, s * x)


def _gat_heads(mask, Tp, ft_all, al, ar, eec, few, brow, D):
    acc = None
    for hh in range(_H):
        ft = ft_all[:, :, hh * D:(hh + 1) * D]
        ft_s = ft[:, :_J, :]
        el = jnp.sum(ft_s * al[hh][None, None, :], axis=-1)
        er = jnp.sum(ft * ar[hh][None, None, :], axis=-1)
        ee = Tp * eec[0:1, hh:hh + 1][:, :, None]
        logits = _lrelu(el[:, :, None] + er[:, None, :] + ee, 0.2)
        logits = jnp.where(mask, logits, -1e9)
        mx = jnp.max(logits, axis=1, keepdims=True)
        ex = jnp.where(mask, jnp.exp(logits - mx), 0.0)
        den = jnp.sum(ex, axis=1, keepdims=True)
        alpha = ex / jnp.where(den > 0, den, 1.0)
        out = lax.dot_general(
            alpha, ft_s,
            dimension_numbers=(((1,), (1,)), ((0,), (0,))),
            preferred_element_type=jnp.float32)
        eagg = jnp.sum(alpha * Tp, axis=1)
        o = out + eagg[:, :, None] * few[hh][None, None, :] + brow[hh][None, None, :]
        o = _lrelu(o, 0.01)
        acc = o if acc is None else acc + o
    return acc * (1.0 / _H)


def _gnn_body(G_ref, T_ref, h_ref, L_ref, wpn_ref,
              W0g_ref, c0_ref, al0_ref, ar0_ref, eec0_ref, few0_ref, b0_ref,
              W1_ref, al1_ref, ar1_ref, eec1_ref, few1_ref, b1_ref,
              h1_ref):
    G = G_ref[...]
    mask = G > 0.0
    zpadT = jnp.zeros(G.shape[:2] + (_N - _J,), jnp.float32)
    Tp = jnp.concatenate([T_ref[...], zpadT], axis=2)
    bb = G.shape[0]

    zpadn = jnp.zeros((bb, _N - _J), jnp.float32)
    nm = (lax.broadcasted_iota(jnp.int32, (bb, _N), 1) < _J).astype(jnp.float32)
    wpn = wpn_ref[...]
    f0 = jnp.concatenate([h_ref[...], zpadn], axis=1)
    f1 = jnp.concatenate([L_ref[...], zpadn], axis=1)
    f2 = wpn[:, 0:1] * nm
    f3 = wpn[:, 1:2] * nm
    f4 = wpn[:, 2:3] * nm
    mu = (f0 + f1 + f2 + f3 + f4) * 0.2
    d0, d1, d2, d3, d4 = f0 - mu, f1 - mu, f2 - mu, f3 - mu, f4 - mu
    var = (d0 * d0 + d1 * d1 + d2 * d2 + d3 * d3 + d4 * d4) * 0.2
    rstd = lax.rsqrt(var + 1e-5)

    W0g = W0g_ref[...]
    ft0 = jnp.broadcast_to(c0_ref[...][None], (bb, _N, _H * _F0))
    for i, d in enumerate((d0, d1, d2, d3, d4)):
        ft0 = ft0 + (d * rstd)[:, :, None] * W0g[i][None, None, :]

    h0 = _gat_heads(mask, Tp, ft0,
                    al0_ref[...], ar0_ref[...], eec0_ref[...],
                    few0_ref[...], b0_ref[...], _F0)

    ft1 = lax.dot_general(h0, W1_ref[...],
                          dimension_numbers=(((2,), (0,)), ((), ())),
                          preferred_element_type=jnp.float32)

    h1 = _gat_heads(mask, Tp, ft1,
                    al1_ref[...], ar1_ref[...], eec1_ref[...],
                    few1_ref[...], b1_ref[...], _ED)
    h1_ref[...] = h1


def _final_body(x_ref, w_ref, bl_ref, o_ref, *, kb):
    k = pl.program_id(0)
    acc = jnp.dot(x_ref[...], w_ref[...], preferred_element_type=jnp.float32)

    @pl.when(k == 0)
    def _():
        o_ref[...] = acc

    @pl.when(k > 0)
    def _():
        o_ref[...] = o_ref[...] + acc

    @pl.when(k == kb - 1)
    def _():
        o_ref[...] = _lrelu(o_ref[...] + bl_ref[...], 0.01)


def kernel(Graph, norm_h, norm_L, norm_W, norm_P, norm_N, T, ln_g, ln_b,
           W0, We0, al0, ar0, ae0, b0, W1, We1, al1, ar1, ae1, b1, Wl, bl):
    BS = Graph.shape[0]
    G = Graph.reshape(BS, _J, _N)
    wpn = jnp.concatenate([norm_W, norm_P, norm_N], axis=-1)

    W0g = W0 * ln_g[:, None]
    c0 = (ln_b @ W0).reshape(1, _H * _F0)
    few0 = We0.reshape(_H, _F0)
    eec0 = jnp.sum(few0 * ae0, axis=-1).reshape(1, _H)
    b0r = b0.reshape(_H, _F0)
    few1 = We1.reshape(_H, _ED)
    eec1 = jnp.sum(few1 * ae1, axis=-1).reshape(1, _H)
    b1r = b1.reshape(_H, _ED)

    nsteps = BS // _BB
    const = lambda *shape: pl.BlockSpec(shape, lambda i: (0,) * len(shape))
    h1 = pl.pallas_call(
        _gnn_body,
        grid=(nsteps,),
        in_specs=[
            pl.BlockSpec((_BB, _J, _N), lambda i: (i, 0, 0)),
            pl.BlockSpec((_BB, _J, _J), lambda i: (i, 0, 0)),
            pl.BlockSpec((_BB, _J), lambda i: (i, 0)),
            pl.BlockSpec((_BB, _J), lambda i: (i, 0)),
            pl.BlockSpec((_BB, 3), lambda i: (i, 0)),
            const(5, _H * _F0),
            const(1, _H * _F0),
            const(_H, _F0), const(_H, _F0),
            const(1, _H),
            const(_H, _F0), const(_H, _F0),
            const(_F0, _H * _ED),
            const(_H, _ED), const(_H, _ED),
            const(1, _H),
            const(_H, _ED), const(_H, _ED),
        ],
        out_specs=pl.BlockSpec((_BB, _N, _ED), lambda i: (i, 0, 0)),
        out_shape=jax.ShapeDtypeStruct((BS, _N, _ED), jnp.float32),
    )(G, T, norm_h, norm_L, wpn, W0g, c0, al0, ar0, eec0, few0, b0r,
      W1, al1, ar1, eec1, few1, b1r)

    X2 = h1.reshape(BS, _N * _ED)
    KB = 8
    BK = (_N * _ED) // KB
    import functools
    out = pl.pallas_call(
        functools.partial(_final_body, kb=KB),
        grid=(KB,),
        in_specs=[
            pl.BlockSpec((BS, BK), lambda k: (0, k)),
            pl.BlockSpec((BK, _ED), lambda k: (k, 0)),
            pl.BlockSpec((1, _ED), lambda k: (0, 0)),
        ],
        out_specs=pl.BlockSpec((BS, _ED), lambda k: (0, 0)),
        out_shape=jax.ShapeDtypeStruct((BS, _ED), jnp.float32),
    )(X2, Wl, bl.reshape(1, _ED))
    return out
